# baseline (speedup 1.0000x reference)
import jax
import jax.numpy as jnp
from jax import lax
from jax.experimental import pallas as pl
from jax.experimental.pallas import tpu as pltpu
from jax.experimental.pallas import tpu_sc as plsc

N = 10000
E = 160000
T = 320000
D = 9
HID = 128
CV = 32

NC = 2
NS = 16
NW = NC * NS
CH = 128

N_TCH = T // CH
N_ECH = E // CH

RANGES_PER_CORE = 2
RNG = E // (NC * RANGES_PER_CORE)
SLAB = 100
TRASH = RNG

EB2 = 640
EB3 = 640


def _sigmoid_tc(v):
    return 0.5 * jnp.tanh(0.5 * v) + 0.5


def _silu(v):
    return v * _sigmoid_tc(v)


def _vrsqrt(x):
    i = plsc.bitcast(x, jnp.int32)
    i = jnp.int32(0x5F3759DF) - (i >> 1)
    y = plsc.bitcast(i, jnp.float32)
    for _ in range(3):
        y = y * (jnp.float32(1.5) - jnp.float32(0.5) * x * y * y)
    return y


def _tc1_body(x_ref, wxh_ref, bxh_ref, wxk_ref, bxk_ref, wa_ref, wb_ref,
              sig_ref, a_ref, b_ref):
    x = x_ref[...]
    xh = jnp.dot(x, wxh_ref[...], preferred_element_type=jnp.float32) + bxh_ref[...]
    xk = jnp.dot(x, wxk_ref[...], preferred_element_type=jnp.float32) + bxk_ref[...]
    sig_ref[...] = _sigmoid_tc(xk)
    sx = _silu(xh)
    a_ref[...] = jnp.dot(sx, wa_ref[...], preferred_element_type=jnp.float32)
    b_ref[...] = jnp.dot(sx, wb_ref[...], preferred_element_type=jnp.float32)


def _tc1(x, wxh, bxh, wxk, bxk, wa, wb):
    bn = 1000
    grid = (N // bn,)
    w_spec = lambda shape: pl.BlockSpec(shape, lambda i: (0, 0))
    return pl.pallas_call(
        _tc1_body,
        grid=grid,
        in_specs=[
            pl.BlockSpec((bn, HID), lambda i: (i, 0)),
            w_spec((HID, CV)), w_spec((1, CV)),
            w_spec((HID, CV)), w_spec((1, CV)),
            w_spec((CV, CV)), w_spec((CV, CV)),
        ],
        out_specs=[
            pl.BlockSpec((bn, CV), lambda i: (i, 0)),
            pl.BlockSpec((bn, CV), lambda i: (i, 0)),
            pl.BlockSpec((bn, CV), lambda i: (i, 0)),
        ],
        out_shape=[
            jax.ShapeDtypeStruct((N, CV), jnp.float32),
            jax.ShapeDtypeStruct((N, CV), jnp.float32),
            jax.ShapeDtypeStruct((N, CV), jnp.float32),
        ],
    )(x, wxh, bxh, wxk, bxk, wa, wb)


def _tc2_body(cji_ref, robs_ref, shbs_ref, wc1b_ref, wc2bb_ref, g_ref,
              rexp_ref, m_ref, shbs16_ref):
    eb = robs_ref.shape[0]

    cji144 = jnp.reshape(cji_ref[...], (eb, D * 16))
    c1 = jnp.dot(_silu(cji144), wc1b_ref[...],
                 preferred_element_type=jnp.float32)
    ckj = jnp.dot(_silu(c1), wc2bb_ref[...],
                  preferred_element_type=jnp.float32)
    ssb = jnp.dot(ckj * ckj, g_ref[...],
                  preferred_element_type=jnp.float32)
    robs_band = jnp.dot(robs_ref[...], rexp_ref[...],
                        preferred_element_type=jnp.float32)
    mband = ckj * lax.rsqrt(jnp.maximum(ssb, 1e-24)) * robs_band
    m_ref[...] = jnp.concatenate(
        [mband, jnp.zeros((eb, 384 - D * CV), jnp.float32)], axis=1)
    shbs16_ref[...] = jnp.pad(shbs_ref[...], ((0, 0), (0, 16 - D)))


def _tc2(cji144, robs, shbs, wc1b, wc2bb, g, rexp):
    nblk = E // EB2
    tb = T // nblk
    grid = (nblk,)
    w_spec = lambda shape: pl.BlockSpec(shape, lambda i: (0, 0))
    return pl.pallas_call(
        _tc2_body,
        grid=grid,
        in_specs=[
            pl.BlockSpec((EB2, D, 16), lambda i: (i, 0, 0)),
            pl.BlockSpec((EB2, D), lambda i: (i, 0)),
            pl.BlockSpec((tb, D), lambda i: (i, 0)),
            w_spec((D * 16, D * CV)), w_spec((D * CV, D * CV)),
            w_spec((D * CV, D * CV)), w_spec((D, D * CV)),
        ],
        out_specs=[
            pl.BlockSpec((EB2, 384), lambda i: (i, 0)),
            pl.BlockSpec((tb, 16), lambda i: (i, 0)),
        ],
        out_shape=[
            jax.ShapeDtypeStruct((E, 384), jnp.float32),
            jax.ShapeDtypeStruct((T, 16), jnp.float32),
        ],
    )(cji144, robs, shbs, wc1b, wc2bb, g, rexp)


def _sc1_body(m_hbm, shbs_hbm, sig_hbm, kj_hbm, k_hbm, tbw_hbm,
              kj_v0, kj_v1, k_v0, k_v1, shbs_v0, shbs_v1, m_v0, m_v1,
              xkg_v0, xkg_v1, tbw_v, semm0, semm1, semx0, semx1):
    wid = lax.axis_index("s") * NC + lax.axis_index("c")
    n_my = 78 + jnp.where(wid < (N_TCH - 78 * NW), 1, 0)
    kj_v = (kj_v0, kj_v1)
    k_v = (k_v0, k_v1)
    shbs_v = (shbs_v0, shbs_v1)
    m_v = (m_v0, m_v1)
    xkg_v = (xkg_v0, xkg_v1)
    semm = (semm0, semm1)
    semx = (semx0, semx1)

    def fetch(i, b):
        base = (wid + i * NW) * CH
        pltpu.sync_copy(kj_hbm.at[pl.ds(base, CH)], kj_v[b])
        pltpu.sync_copy(k_hbm.at[pl.ds(base, CH)], k_v[b])
        pltpu.sync_copy(shbs_hbm.at[pl.ds(base, CH)], shbs_v[b])
        pltpu.async_copy(m_hbm.at[kj_v[b]], m_v[b], semm[b])
        pltpu.async_copy(sig_hbm.at[k_v[b]], xkg_v[b], semx[b])

    for b in range(2):
        @pl.when(b < n_my)
        def _(b=b):
            fetch(b, b)

    def outer_body(j, _):
        for b in range(2):
            i = 2 * j + b

            @pl.when(i < n_my)
            def _(i=i, b=b):
                pltpu.make_async_copy(m_hbm.at[kj_v[b]], m_v[b],
                                      semm[b]).wait()
                pltpu.make_async_copy(sig_hbm.at[k_v[b]], xkg_v[b],
                                      semx[b]).wait()

                @plsc.parallel_loop(0, CH, unroll=4)
                def tri_body(t):
                    wrow = shbs_v[b][t, pl.ds(0, 16)]
                    acc0 = m_v[b][t, pl.ds(0, 16)] * wrow[0]
                    acc1 = m_v[b][t, pl.ds(16, 16)] * wrow[0]
                    for d in range(1, D):
                        sd = wrow[d]
                        acc0 = acc0 + m_v[b][t, pl.ds(d * CV, 16)] * sd
                        acc1 = acc1 + m_v[b][t, pl.ds(d * CV + 16, 16)] * sd
                    ss = jnp.sum(acc0 * acc0 + acc1 * acc1)
                    r = _vrsqrt(jnp.full((16,), jnp.maximum(ss, 1e-24),
                                         jnp.float32))
                    tbw_v[t, pl.ds(0, 16)] = acc0 * r * xkg_v[b][t, pl.ds(0, 16)]
                    tbw_v[t, pl.ds(16, 16)] = acc1 * r * xkg_v[b][t, pl.ds(16, 16)]
                base = (wid + i * NW) * CH
                pltpu.sync_copy(tbw_v, tbw_hbm.at[pl.ds(base, CH)])

                @pl.when(i + 2 < n_my)
                def _():
                    fetch(i + 2, b)
        return 0

    lax.fori_loop(0, 40, outer_body, 0)


def _sc1(m_table, shbs16, sigxk, edge_idx_kj, tri_idx_k):
    mesh = plsc.VectorSubcoreMesh(core_axis_name="c", subcore_axis_name="s",
                                  num_cores=NC, num_subcores=NS)
    f = pl.kernel(
        _sc1_body,
        out_type=jax.ShapeDtypeStruct((T, CV), jnp.float32),
        mesh=mesh,
        compiler_params=pltpu.CompilerParams(needs_layout_passes=False,
                                             use_tc_tiling_on_sc=False),
        scratch_types=[
            pltpu.VMEM((CH,), jnp.int32),
            pltpu.VMEM((CH,), jnp.int32),
            pltpu.VMEM((CH,), jnp.int32),
            pltpu.VMEM((CH,), jnp.int32),
            pltpu.VMEM((CH, 16), jnp.float32),
            pltpu.VMEM((CH, 16), jnp.float32),
            pltpu.VMEM((CH, 384), jnp.float32),
            pltpu.VMEM((CH, 384), jnp.float32),
            pltpu.VMEM((CH, CV), jnp.float32),
            pltpu.VMEM((CH, CV), jnp.float32),
            pltpu.VMEM((CH, CV), jnp.float32),
            pltpu.SemaphoreType.DMA,
            pltpu.SemaphoreType.DMA,
            pltpu.SemaphoreType.DMA,
            pltpu.SemaphoreType.DMA,
        ],
    )
    return f(m_table, shbs16, sigxk, edge_idx_kj, tri_idx_k)


def _scg_body(a_hbm, b_hbm, ii_hbm, jj_hbm, nm1_hbm,
              ii_v, jj_v, av_v, bv_v, sem1, sem2):
    wid = lax.axis_index("s") * NC + lax.axis_index("c")
    n_my = 39 + jnp.where(wid < (N_ECH - 39 * NW), 1, 0)

    def chunk_body(i, _):
        base = (wid + i * NW) * CH
        pltpu.sync_copy(ii_hbm.at[pl.ds(base, CH)], ii_v)
        pltpu.sync_copy(jj_hbm.at[pl.ds(base, CH)], jj_v)
        cp1 = pltpu.async_copy(a_hbm.at[ii_v], av_v, sem1)
        cp2 = pltpu.async_copy(b_hbm.at[jj_v], bv_v, sem2)
        cp1.wait()
        cp2.wait()

        @plsc.parallel_loop(0, CH, unroll=4)
        def row_body(t):
            av_v[t, pl.ds(0, 16)] = av_v[t, pl.ds(0, 16)] + bv_v[t, pl.ds(0, 16)]
            av_v[t, pl.ds(16, 16)] = av_v[t, pl.ds(16, 16)] + bv_v[t, pl.ds(16, 16)]
        pltpu.sync_copy(av_v, nm1_hbm.at[pl.ds(base, CH), pl.ds(0, CV)])
        return 0

    lax.fori_loop(0, n_my, chunk_body, 0)


def _scg(a, b, idx_i, idx_j):
    mesh = plsc.VectorSubcoreMesh(core_axis_name="c", subcore_axis_name="s",
                                  num_cores=NC, num_subcores=NS)
    f = pl.kernel(
        _scg_body,
        out_type=jax.ShapeDtypeStruct((E, 128), jnp.float32),
        mesh=mesh,
        compiler_params=pltpu.CompilerParams(needs_layout_passes=False,
                                             use_tc_tiling_on_sc=False),
        scratch_types=[
            pltpu.VMEM((CH,), jnp.int32),
            pltpu.VMEM((CH,), jnp.int32),
            pltpu.VMEM((CH, CV), jnp.float32),
            pltpu.VMEM((CH, CV), jnp.float32),
            pltpu.SemaphoreType.DMA,
            pltpu.SemaphoreType.DMA,
        ],
    )
    return f(a, b, idx_i, idx_j)


def _sc2_body(tbw_hbm, ji_hbm, agg_hbm,
              ji_v0, ji_v1, idx_v, rows_v0, rows_v1, zslab_v, sp_ref,
              semj0, semj1, semr0, semr1):
    c = lax.axis_index("c")
    s = lax.axis_index("s")
    ji_v = (ji_v0, ji_v1)
    rows_v = (rows_v0, rows_v1)
    semj = (semj0, semj1)
    semr = (semr0, semr1)

    def zrow(t, _):
        zslab_v[t, pl.ds(0, 16)] = jnp.zeros((16,), jnp.float32)
        zslab_v[t, pl.ds(16, 16)] = jnp.zeros((16,), jnp.float32)
        return 0
    lax.fori_loop(0, SLAB, zrow, 0)

    rows_per_tile = RNG // NS
    n_slabs = rows_per_tile // SLAB

    for r in range(RANGES_PER_CORE):
        base_e = (c * RANGES_PER_CORE + r) * RNG

        def zero_body(i, _):
            pltpu.sync_copy(zslab_v,
                            sp_ref.at[pl.ds(s * rows_per_tile + i * SLAB, SLAB)])
            return 0
        lax.fori_loop(0, n_slabs, zero_body, 0)

        @pl.when(s == 0)
        def _():
            pltpu.sync_copy(zslab_v.at[pl.ds(0, 8)], sp_ref.at[pl.ds(RNG, 8)])

        plsc.subcore_barrier()

        n_my = 156 + jnp.where(s < (N_TCH - 156 * NS), 1, 0)

        def fetch(i, b):
            base = (s + i * NS) * CH
            pltpu.async_copy(ji_hbm.at[pl.ds(base, CH)], ji_v[b], semj[b])
            pltpu.async_copy(tbw_hbm.at[pl.ds(base, CH)], rows_v[b], semr[b])

        for b in range(2):
            @pl.when(b < n_my)
            def _(b=b):
                fetch(b, b)

        def scan_outer(j, _):
            for b in range(2):
                i = 2 * j + b

                @pl.when(i < n_my)
                def _(i=i, b=b):
                    base = (s + i * NS) * CH
                    pltpu.make_async_copy(ji_hbm.at[pl.ds(base, CH)],
                                          ji_v[b], semj[b]).wait()
                    pltpu.make_async_copy(tbw_hbm.at[pl.ds(base, CH)],
                                          rows_v[b], semr[b]).wait()
                    for g in range(CH // 16):
                        v = ji_v[b][pl.ds(g * 16, 16)] - base_e
                        ok = (v >= 0) & (v < RNG)
                        idx_v[pl.ds(g * 16, 16)] = jnp.where(ok, v, TRASH)
                    pltpu.sync_copy(rows_v[b], sp_ref.at[idx_v], add=True)

                    @pl.when(i + 2 < n_my)
                    def _():
                        fetch(i + 2, b)
            return 0

        lax.fori_loop(0, 79, scan_outer, 0)
        plsc.subcore_barrier()

        def wb_body(i, _):
            off = s * rows_per_tile + i * SLAB
            pltpu.sync_copy(sp_ref.at[pl.ds(off, SLAB)],
                            rows_v0.at[pl.ds(0, SLAB)])
            pltpu.sync_copy(rows_v0.at[pl.ds(0, SLAB)],
                            agg_hbm.at[pl.ds(base_e + off, SLAB), pl.ds(0, CV)])
            return 0
        lax.fori_loop(0, n_slabs, wb_body, 0)
        plsc.subcore_barrier()


def _sc2(tbw, edge_idx_ji):
    mesh = plsc.VectorSubcoreMesh(core_axis_name="c", subcore_axis_name="s",
                                  num_cores=NC, num_subcores=NS)
    f = pl.kernel(
        _sc2_body,
        out_type=jax.ShapeDtypeStruct((E, 128), jnp.float32),
        mesh=mesh,
        compiler_params=pltpu.CompilerParams(needs_layout_passes=False,
                                             use_tc_tiling_on_sc=False),
        scratch_types=[
            pltpu.VMEM((CH,), jnp.int32),
            pltpu.VMEM((CH,), jnp.int32),
            pltpu.VMEM((CH,), jnp.int32),
            pltpu.VMEM((CH, CV), jnp.float32),
            pltpu.VMEM((CH, CV), jnp.float32),
            pltpu.VMEM((SLAB, CV), jnp.float32),
            pltpu.VMEM_SHARED((RNG + 8, CV), jnp.float32),
            pltpu.SemaphoreType.DMA,
            pltpu.SemaphoreType.DMA,
            pltpu.SemaphoreType.DMA,
            pltpu.SemaphoreType.DMA,
        ],
    )
    return f(tbw, edge_idx_ji)


def _tc3_body(cji_ref, robs_ref, agg_ref, nm1_ref, wc1b_ref, wc2ab_ref,
              g_ref, rexp_ref, t32_ref, wt_ref, bt_ref, bn1_ref, wn2_ref,
              bn2_ref, msg_ref):
    eb = agg_ref.shape[0]

    tbw = jnp.dot(_silu(agg_ref[..., :CV]), wt_ref[...],
                  preferred_element_type=jnp.float32) + bt_ref[...]
    scale_band = 1.0 + jnp.dot(tbw, t32_ref[...],
                               preferred_element_type=jnp.float32)

    cji144 = jnp.reshape(cji_ref[...], (eb, D * 16))
    c1 = jnp.dot(_silu(cji144), wc1b_ref[...],
                 preferred_element_type=jnp.float32)
    cji2 = jnp.dot(_silu(c1), wc2ab_ref[...],
                   preferred_element_type=jnp.float32)
    c2 = cji2 * scale_band
    ssb = jnp.dot(c2 * c2, g_ref[...], preferred_element_type=jnp.float32)
    robs_band = jnp.dot(robs_ref[...], rexp_ref[...],
                        preferred_element_type=jnp.float32)
    weighted = c2 * lax.rsqrt(jnp.maximum(ssb, 1e-24)) * robs_band
    lcao = lax.dot_general(weighted, t32_ref[...], (((1,), (1,)), ((), ())),
                           preferred_element_type=jnp.float32)
    ss2 = jnp.sum(lcao * lcao, axis=1, keepdims=True)
    lcao_n = lcao * lax.rsqrt(jnp.maximum(ss2, 1e-24))

    nm = jnp.dot(_silu(nm1_ref[..., :CV] + bn1_ref[...]), wn2_ref[...],
                 preferred_element_type=jnp.float32) + bn2_ref[...]
    msg = lcao_n * nm
    msg_ref[...] = jnp.concatenate(
        [msg, jnp.zeros((eb, 128 - CV), jnp.float32)], axis=1)


def _tc3(cji144, robs, agg, nm1, wc1b, wc2ab, g, rexp, t32, wt, bt, bn1,
         wn2, bn2):
    eb = EB3
    nblk = E // eb
    grid = (nblk,)
    w_spec = lambda shape: pl.BlockSpec(shape, lambda i: (0, 0))
    return pl.pallas_call(
        _tc3_body,
        grid=grid,
        in_specs=[
            pl.BlockSpec((eb, D, 16), lambda i: (i, 0, 0)),
            pl.BlockSpec((eb, D), lambda i: (i, 0)),
            pl.BlockSpec((eb, 128), lambda i: (i, 0)),
            pl.BlockSpec((eb, 128), lambda i: (i, 0)),
            w_spec((D * 16, D * CV)), w_spec((D * CV, D * CV)),
            w_spec((D * CV, D * CV)), w_spec((D, D * CV)),
            w_spec((CV, D * CV)),
            w_spec((CV, CV)), w_spec((1, CV)), w_spec((1, CV)),
            w_spec((CV, CV)), w_spec((1, CV)),
        ],
        out_specs=pl.BlockSpec((eb, 128), lambda i: (i, 0)),
        out_shape=jax.ShapeDtypeStruct((E, 128), jnp.float32),
    )(cji144, robs, agg, nm1, wc1b, wc2ab, g, rexp, t32, wt, bt, bn1,
      wn2, bn2)


def _sc3_body(msg_hbm, ii_hbm, part_hbm,
              ii_v, rows_v, zslab_v, sp_ref):
    c = lax.axis_index("c")
    s = lax.axis_index("s")
    wid = s * NC + c

    def zrow(t, _):
        zslab_v[t, pl.ds(0, 16)] = jnp.zeros((16,), jnp.float32)
        zslab_v[t, pl.ds(16, 16)] = jnp.zeros((16,), jnp.float32)
        return 0
    lax.fori_loop(0, 125, zrow, 0)

    rows_per_tile = N // NS

    def zero_body(i, _):
        pltpu.sync_copy(zslab_v, sp_ref.at[pl.ds(s * rows_per_tile + i * 125, 125)])
        return 0
    lax.fori_loop(0, rows_per_tile // 125, zero_body, 0)
    plsc.subcore_barrier()

    n_my = 39 + jnp.where(wid < (N_ECH - 39 * NW), 1, 0)

    def scan_body(i, _):
        base = (wid + i * NW) * CH
        pltpu.sync_copy(ii_hbm.at[pl.ds(base, CH)], ii_v)
        pltpu.sync_copy(msg_hbm.at[pl.ds(base, CH), pl.ds(0, CV)], rows_v)
        pltpu.sync_copy(rows_v, sp_ref.at[ii_v], add=True)
        return 0

    lax.fori_loop(0, n_my, scan_body, 0)
    plsc.subcore_barrier()

    def wb_body(i, _):
        off = s * rows_per_tile + i * 125
        pltpu.sync_copy(sp_ref.at[pl.ds(off, 125)], rows_v.at[pl.ds(0, 125)])
        pltpu.sync_copy(rows_v.at[pl.ds(0, 125)],
                        part_hbm.at[pl.ds(c * N + off, 125), pl.ds(0, CV)])
        return 0
    lax.fori_loop(0, rows_per_tile // 125, wb_body, 0)


def _sc3(msg, idx_i):
    mesh = plsc.VectorSubcoreMesh(core_axis_name="c", subcore_axis_name="s",
                                  num_cores=NC, num_subcores=NS)
    f = pl.kernel(
        _sc3_body,
        out_type=jax.ShapeDtypeStruct((NC * N, 128), jnp.float32),
        mesh=mesh,
        compiler_params=pltpu.CompilerParams(needs_layout_passes=False,
                                             use_tc_tiling_on_sc=False),
        scratch_types=[
            pltpu.VMEM((CH,), jnp.int32),
            pltpu.VMEM((CH, CV), jnp.float32),
            pltpu.VMEM((125, CV), jnp.float32),
            pltpu.VMEM_SHARED((N, CV), jnp.float32),
        ],
    )
    return f(msg, idx_i)


def _tc4_body(x_ref, p0_ref, p1_ref, wna_ref, out_ref):
    agg = p0_ref[..., :CV] + p1_ref[..., :CV]
    out_ref[...] = x_ref[...] + jnp.dot(agg, wna_ref[...],
                                        preferred_element_type=jnp.float32)


def _tc4(x, parts, wna):
    bn = 1000
    nb = N // bn
    grid = (nb,)
    return pl.pallas_call(
        _tc4_body,
        grid=grid,
        in_specs=[
            pl.BlockSpec((bn, HID), lambda i: (i, 0)),
            pl.BlockSpec((bn, 128), lambda i: (i, 0)),
            pl.BlockSpec((bn, 128), lambda i: (i + nb, 0)),
            pl.BlockSpec((CV, HID), lambda i: (0, 0)),
        ],
        out_specs=pl.BlockSpec((bn, HID), lambda i: (i, 0)),
        out_shape=jax.ShapeDtypeStruct((N, HID), jnp.float32),
    )(x, parts, parts, wna)


def kernel(x, cji, outer_mask, robs, shbs, idx_i, idx_j, tri_idx_k,
           edge_idx_kj, edge_idx_ji, W_nb, b_nb, W_c1, W_c2, W_t, b_t,
           W_n1, b_n1, W_n2, b_n2, W_na):
    del outer_mask
    idx_i = idx_i.astype(jnp.int32)
    idx_j = idx_j.astype(jnp.int32)
    tri_idx_k = tri_idx_k.astype(jnp.int32)
    edge_idx_kj = edge_idx_kj.astype(jnp.int32)
    edge_idx_ji = edge_idx_ji.astype(jnp.int32)

    wxh, wxk = W_nb[:, :CV], W_nb[:, CV:]
    bxh, bxk = b_nb[:CV].reshape(1, CV), b_nb[CV:].reshape(1, CV)
    wc2a, wc2b = W_c2[:, :CV], W_c2[:, CV:]
    wn1a, wn1b = W_n1[:CV, :], W_n1[CV:, :]
    bt = b_t.reshape(1, CV)
    bn1 = b_n1.reshape(1, CV)
    bn2 = b_n2.reshape(1, CV)

    ii = jnp.arange(D * CV, dtype=jnp.int32)
    dgrp = ii // CV
    eye_d = lambda a, b: (a[:, None] == b[None, :]).astype(jnp.float32)
    blkdiag16 = jnp.zeros((D * 16, D * CV), jnp.float32)
    for d in range(D):
        blkdiag16 = blkdiag16.at[d * 16:(d + 1) * 16,
                                 d * CV:(d + 1) * CV].set(W_c1)
    wc2ab = jnp.zeros((D * CV, D * CV), jnp.float32)
    wc2bb = jnp.zeros((D * CV, D * CV), jnp.float32)
    for d in range(D):
        sl = slice(d * CV, (d + 1) * CV)
        wc2ab = wc2ab.at[sl, sl].set(wc2a)
        wc2bb = wc2bb.at[sl, sl].set(wc2b)
    g = eye_d(dgrp, dgrp)
    rexp = eye_d(jnp.arange(D, dtype=jnp.int32), dgrp)
    t32 = eye_d(jnp.arange(CV, dtype=jnp.int32), ii % CV)

    sigxk, a_tab, b_tab = _tc1(x, wxh, bxh, wxk, bxk, wn1a, wn1b)
    m_table, shbs16 = _tc2(cji, robs, shbs, blkdiag16, wc2bb, g, rexp)

    tbw = _sc1(m_table, shbs16, sigxk, edge_idx_kj, tri_idx_k)
    nm1 = _scg(a_tab, b_tab, idx_i, idx_j)
    agg = _sc2(tbw, edge_idx_ji)

    msg = _tc3(cji, robs, agg, nm1, blkdiag16, wc2ab, g, rexp, t32,
               W_t, bt, bn1, W_n2, bn2)
    parts = _sc3(msg, idx_i)
    return _tc4(x, parts, W_na)


# speedup vs baseline: 1.4751x; 1.4751x over previous
import jax
import jax.numpy as jnp
from jax import lax
from jax.experimental import pallas as pl
from jax.experimental.pallas import tpu as pltpu
from jax.experimental.pallas import tpu_sc as plsc

N = 10000
E = 160000
T = 320000
D = 9
HID = 128
CV = 32

NC = 2
NS = 16
NW = NC * NS
CH = 128

N_TCH = T // CH
N_ECH = E // CH

RANGES_PER_CORE = 2
RNG = E // (NC * RANGES_PER_CORE)
SLAB = 100
TRASH = RNG

EB2 = 640
EB3 = 640


def _sigmoid_tc(v):
    return 0.5 * jnp.tanh(0.5 * v) + 0.5


def _silu(v):
    return v * _sigmoid_tc(v)


def _vrsqrt(x):
    i = plsc.bitcast(x, jnp.int32)
    i = jnp.int32(0x5F3759DF) - (i >> 1)
    y = plsc.bitcast(i, jnp.float32)
    for _ in range(3):
        y = y * (jnp.float32(1.5) - jnp.float32(0.5) * x * y * y)
    return y


def _tc1_body(x_ref, wxh_ref, bxh_ref, wxk_ref, bxk_ref, wa_ref, wb_ref,
              sig_ref, a_ref, b_ref):
    x = x_ref[...]
    xh = jnp.dot(x, wxh_ref[...], preferred_element_type=jnp.float32) + bxh_ref[...]
    xk = jnp.dot(x, wxk_ref[...], preferred_element_type=jnp.float32) + bxk_ref[...]
    sig_ref[...] = _sigmoid_tc(xk)
    sx = _silu(xh)
    a_ref[...] = jnp.dot(sx, wa_ref[...], preferred_element_type=jnp.float32)
    b_ref[...] = jnp.dot(sx, wb_ref[...], preferred_element_type=jnp.float32)


def _tc1(x, wxh, bxh, wxk, bxk, wa, wb):
    bn = 1000
    grid = (N // bn,)
    w_spec = lambda shape: pl.BlockSpec(shape, lambda i: (0, 0))
    return pl.pallas_call(
        _tc1_body,
        grid=grid,
        in_specs=[
            pl.BlockSpec((bn, HID), lambda i: (i, 0)),
            w_spec((HID, CV)), w_spec((1, CV)),
            w_spec((HID, CV)), w_spec((1, CV)),
            w_spec((CV, CV)), w_spec((CV, CV)),
        ],
        out_specs=[
            pl.BlockSpec((bn, CV), lambda i: (i, 0)),
            pl.BlockSpec((bn, CV), lambda i: (i, 0)),
            pl.BlockSpec((bn, CV), lambda i: (i, 0)),
        ],
        out_shape=[
            jax.ShapeDtypeStruct((N, CV), jnp.float32),
            jax.ShapeDtypeStruct((N, CV), jnp.float32),
            jax.ShapeDtypeStruct((N, CV), jnp.float32),
        ],
    )(x, wxh, bxh, wxk, bxk, wa, wb)


def _tc2_body(cji_ref, robs_ref, wc1b_ref, wc2bb_ref, g_ref,
              rexp_ref, m_ref):
    eb = robs_ref.shape[0]

    c1 = jnp.dot(_silu(cji_ref[...]), wc1b_ref[...],
                 preferred_element_type=jnp.float32)
    ckj = jnp.dot(_silu(c1), wc2bb_ref[...],
                  preferred_element_type=jnp.float32)
    ssb = jnp.dot(ckj * ckj, g_ref[...],
                  preferred_element_type=jnp.float32)
    robs_band = jnp.dot(robs_ref[...], rexp_ref[...],
                        preferred_element_type=jnp.float32)
    mband = ckj * lax.rsqrt(jnp.maximum(ssb, 1e-24)) * robs_band
    m_ref[...] = jnp.concatenate(
        [mband, jnp.zeros((eb, 384 - D * CV), jnp.float32)], axis=1)


def _tc2(cji144, robs, wc1b, wc2bb, g, rexp):
    nblk = E // EB2
    grid = (nblk,)
    w_spec = lambda shape: pl.BlockSpec(shape, lambda i: (0, 0))
    return pl.pallas_call(
        _tc2_body,
        grid=grid,
        in_specs=[
            pl.BlockSpec((EB2, D * 16), lambda i: (i, 0)),
            pl.BlockSpec((EB2, D), lambda i: (i, 0)),
            w_spec((D * 16, D * CV)), w_spec((D * CV, D * CV)),
            w_spec((D * CV, D * CV)), w_spec((D, D * CV)),
        ],
        out_specs=pl.BlockSpec((EB2, 384), lambda i: (i, 0)),
        out_shape=jax.ShapeDtypeStruct((E, 384), jnp.float32),
    )(cji144, robs, wc1b, wc2bb, g, rexp)


def _sc1_body(m_hbm, shbs_hbm, sig_hbm, kj_hbm, k_hbm, tbw_hbm,
              kj_v0, kj_v1, k_v0, k_v1, shbs_v0, shbs_v1, m_v0, m_v1,
              xkg_v0, xkg_v1, tbw_v, semm0, semm1, semx0, semx1):
    wid = lax.axis_index("s") * NC + lax.axis_index("c")
    n_my = 78 + jnp.where(wid < (N_TCH - 78 * NW), 1, 0)
    kj_v = (kj_v0, kj_v1)
    k_v = (k_v0, k_v1)
    shbs_v = (shbs_v0, shbs_v1)
    m_v = (m_v0, m_v1)
    xkg_v = (xkg_v0, xkg_v1)
    semm = (semm0, semm1)
    semx = (semx0, semx1)

    def fetch(i, b):
        base = (wid + i * NW) * CH
        pltpu.sync_copy(kj_hbm.at[pl.ds(base, CH)], kj_v[b])
        pltpu.sync_copy(k_hbm.at[pl.ds(base, CH)], k_v[b])
        pltpu.sync_copy(shbs_hbm.at[pl.ds(base * D, CH * D)], shbs_v[b].at[pl.ds(0, CH * D)])
        pltpu.async_copy(m_hbm.at[kj_v[b]], m_v[b], semm[b])
        pltpu.async_copy(sig_hbm.at[k_v[b]], xkg_v[b], semx[b])

    for b in range(2):
        @pl.when(b < n_my)
        def _(b=b):
            fetch(b, b)

    def outer_body(j, _):
        for b in range(2):
            i = 2 * j + b

            @pl.when(i < n_my)
            def _(i=i, b=b):
                pltpu.make_async_copy(m_hbm.at[kj_v[b]], m_v[b],
                                      semm[b]).wait()
                pltpu.make_async_copy(sig_hbm.at[k_v[b]], xkg_v[b],
                                      semx[b]).wait()

                @plsc.parallel_loop(0, CH, unroll=4)
                def tri_body(t):
                    wrow = shbs_v[b][pl.ds(t * D, 16)]
                    acc0 = m_v[b][t, pl.ds(0, 16)] * wrow[0]
                    acc1 = m_v[b][t, pl.ds(16, 16)] * wrow[0]
                    for d in range(1, D):
                        sd = wrow[d]
                        acc0 = acc0 + m_v[b][t, pl.ds(d * CV, 16)] * sd
                        acc1 = acc1 + m_v[b][t, pl.ds(d * CV + 16, 16)] * sd
                    ss = jnp.sum(acc0 * acc0 + acc1 * acc1)
                    r = _vrsqrt(jnp.full((16,), jnp.maximum(ss, 1e-24),
                                         jnp.float32))
                    tbw_v[t, pl.ds(0, 16)] = acc0 * r * xkg_v[b][t, pl.ds(0, 16)]
                    tbw_v[t, pl.ds(16, 16)] = acc1 * r * xkg_v[b][t, pl.ds(16, 16)]
                base = (wid + i * NW) * CH
                pltpu.sync_copy(tbw_v, tbw_hbm.at[pl.ds(base, CH)])

                @pl.when(i + 2 < n_my)
                def _():
                    fetch(i + 2, b)
        return 0

    lax.fori_loop(0, 40, outer_body, 0)


def _sc1(m_table, shbs16, sigxk, edge_idx_kj, tri_idx_k):
    mesh = plsc.VectorSubcoreMesh(core_axis_name="c", subcore_axis_name="s",
                                  num_cores=NC, num_subcores=NS)
    f = pl.kernel(
        _sc1_body,
        out_type=jax.ShapeDtypeStruct((T, CV), jnp.float32),
        mesh=mesh,
        compiler_params=pltpu.CompilerParams(needs_layout_passes=False,
                                             use_tc_tiling_on_sc=False),
        scratch_types=[
            pltpu.VMEM((CH,), jnp.int32),
            pltpu.VMEM((CH,), jnp.int32),
            pltpu.VMEM((CH,), jnp.int32),
            pltpu.VMEM((CH,), jnp.int32),
            pltpu.VMEM((CH * D + 16, ), jnp.float32),
            pltpu.VMEM((CH * D + 16, ), jnp.float32),
            pltpu.VMEM((CH, 384), jnp.float32),
            pltpu.VMEM((CH, 384), jnp.float32),
            pltpu.VMEM((CH, CV), jnp.float32),
            pltpu.VMEM((CH, CV), jnp.float32),
            pltpu.VMEM((CH, CV), jnp.float32),
            pltpu.SemaphoreType.DMA,
            pltpu.SemaphoreType.DMA,
            pltpu.SemaphoreType.DMA,
            pltpu.SemaphoreType.DMA,
        ],
    )
    return f(m_table, shbs16, sigxk, edge_idx_kj, tri_idx_k)


def _scg_body(a_hbm, b_hbm, ii_hbm, jj_hbm, nm1_hbm,
              ii_v, jj_v, av_v, bv_v, sem1, sem2):
    wid = lax.axis_index("s") * NC + lax.axis_index("c")
    n_my = 39 + jnp.where(wid < (N_ECH - 39 * NW), 1, 0)

    def chunk_body(i, _):
        base = (wid + i * NW) * CH
        pltpu.sync_copy(ii_hbm.at[pl.ds(base, CH)], ii_v)
        pltpu.sync_copy(jj_hbm.at[pl.ds(base, CH)], jj_v)
        cp1 = pltpu.async_copy(a_hbm.at[ii_v], av_v, sem1)
        cp2 = pltpu.async_copy(b_hbm.at[jj_v], bv_v, sem2)
        cp1.wait()
        cp2.wait()

        @plsc.parallel_loop(0, CH, unroll=4)
        def row_body(t):
            av_v[t, pl.ds(0, 16)] = av_v[t, pl.ds(0, 16)] + bv_v[t, pl.ds(0, 16)]
            av_v[t, pl.ds(16, 16)] = av_v[t, pl.ds(16, 16)] + bv_v[t, pl.ds(16, 16)]
        pltpu.sync_copy(av_v, nm1_hbm.at[pl.ds(base, CH), pl.ds(0, CV)])
        return 0

    lax.fori_loop(0, n_my, chunk_body, 0)


def _scg(a, b, idx_i, idx_j):
    mesh = plsc.VectorSubcoreMesh(core_axis_name="c", subcore_axis_name="s",
                                  num_cores=NC, num_subcores=NS)
    f = pl.kernel(
        _scg_body,
        out_type=jax.ShapeDtypeStruct((E, 128), jnp.float32),
        mesh=mesh,
        compiler_params=pltpu.CompilerParams(needs_layout_passes=False,
                                             use_tc_tiling_on_sc=False),
        scratch_types=[
            pltpu.VMEM((CH,), jnp.int32),
            pltpu.VMEM((CH,), jnp.int32),
            pltpu.VMEM((CH, CV), jnp.float32),
            pltpu.VMEM((CH, CV), jnp.float32),
            pltpu.SemaphoreType.DMA,
            pltpu.SemaphoreType.DMA,
        ],
    )
    return f(a, b, idx_i, idx_j)


def _sc2_body(tbw_hbm, ji_hbm, agg_hbm,
              ji_v0, ji_v1, idx_v, rows_v0, rows_v1, zslab_v, sp_ref,
              semj0, semj1, semr0, semr1):
    c = lax.axis_index("c")
    s = lax.axis_index("s")
    ji_v = (ji_v0, ji_v1)
    rows_v = (rows_v0, rows_v1)
    semj = (semj0, semj1)
    semr = (semr0, semr1)

    def zrow(t, _):
        zslab_v[t, pl.ds(0, 16)] = jnp.zeros((16,), jnp.float32)
        zslab_v[t, pl.ds(16, 16)] = jnp.zeros((16,), jnp.float32)
        return 0
    lax.fori_loop(0, SLAB, zrow, 0)

    rows_per_tile = RNG // NS
    n_slabs = rows_per_tile // SLAB

    for r in range(RANGES_PER_CORE):
        base_e = (c * RANGES_PER_CORE + r) * RNG

        def zero_body(i, _):
            pltpu.sync_copy(zslab_v,
                            sp_ref.at[pl.ds(s * rows_per_tile + i * SLAB, SLAB)])
            return 0
        lax.fori_loop(0, n_slabs, zero_body, 0)

        @pl.when(s == 0)
        def _():
            pltpu.sync_copy(zslab_v.at[pl.ds(0, 8)], sp_ref.at[pl.ds(RNG, 8)])

        plsc.subcore_barrier()

        n_my = 156 + jnp.where(s < (N_TCH - 156 * NS), 1, 0)

        def fetch(i, b):
            base = (s + i * NS) * CH
            pltpu.async_copy(ji_hbm.at[pl.ds(base, CH)], ji_v[b], semj[b])
            pltpu.async_copy(tbw_hbm.at[pl.ds(base, CH)], rows_v[b], semr[b])

        for b in range(2):
            @pl.when(b < n_my)
            def _(b=b):
                fetch(b, b)

        def scan_outer(j, _):
            for b in range(2):
                i = 2 * j + b

                @pl.when(i < n_my)
                def _(i=i, b=b):
                    base = (s + i * NS) * CH
                    pltpu.make_async_copy(ji_hbm.at[pl.ds(base, CH)],
                                          ji_v[b], semj[b]).wait()
                    pltpu.make_async_copy(tbw_hbm.at[pl.ds(base, CH)],
                                          rows_v[b], semr[b]).wait()
                    for g in range(CH // 16):
                        v = ji_v[b][pl.ds(g * 16, 16)] - base_e
                        ok = (v >= 0) & (v < RNG)
                        idx_v[pl.ds(g * 16, 16)] = jnp.where(ok, v, TRASH)
                    pltpu.sync_copy(rows_v[b], sp_ref.at[idx_v], add=True)

                    @pl.when(i + 2 < n_my)
                    def _():
                        fetch(i + 2, b)
            return 0

        lax.fori_loop(0, 79, scan_outer, 0)
        plsc.subcore_barrier()

        def wb_body(i, _):
            off = s * rows_per_tile + i * SLAB
            pltpu.sync_copy(sp_ref.at[pl.ds(off, SLAB)],
                            rows_v0.at[pl.ds(0, SLAB)])
            pltpu.sync_copy(rows_v0.at[pl.ds(0, SLAB)],
                            agg_hbm.at[pl.ds(base_e + off, SLAB), pl.ds(0, CV)])
            return 0
        lax.fori_loop(0, n_slabs, wb_body, 0)
        plsc.subcore_barrier()


def _sc2(tbw, edge_idx_ji):
    mesh = plsc.VectorSubcoreMesh(core_axis_name="c", subcore_axis_name="s",
                                  num_cores=NC, num_subcores=NS)
    f = pl.kernel(
        _sc2_body,
        out_type=jax.ShapeDtypeStruct((E, 128), jnp.float32),
        mesh=mesh,
        compiler_params=pltpu.CompilerParams(needs_layout_passes=False,
                                             use_tc_tiling_on_sc=False),
        scratch_types=[
            pltpu.VMEM((CH,), jnp.int32),
            pltpu.VMEM((CH,), jnp.int32),
            pltpu.VMEM((CH,), jnp.int32),
            pltpu.VMEM((CH, CV), jnp.float32),
            pltpu.VMEM((CH, CV), jnp.float32),
            pltpu.VMEM((SLAB, CV), jnp.float32),
            pltpu.VMEM_SHARED((RNG + 8, CV), jnp.float32),
            pltpu.SemaphoreType.DMA,
            pltpu.SemaphoreType.DMA,
            pltpu.SemaphoreType.DMA,
            pltpu.SemaphoreType.DMA,
        ],
    )
    return f(tbw, edge_idx_ji)


def _tc3_body(cji_ref, robs_ref, agg_ref, nm1_ref, wc1b_ref, wc2ab_ref,
              g_ref, rexp_ref, t32_ref, wt_ref, bt_ref, bn1_ref, wn2_ref,
              bn2_ref, msg_ref):
    eb = agg_ref.shape[0]

    tbw = jnp.dot(_silu(agg_ref[..., :CV]), wt_ref[...],
                  preferred_element_type=jnp.float32) + bt_ref[...]
    scale_band = 1.0 + jnp.dot(tbw, t32_ref[...],
                               preferred_element_type=jnp.float32)

    c1 = jnp.dot(_silu(cji_ref[...]), wc1b_ref[...],
                 preferred_element_type=jnp.float32)
    cji2 = jnp.dot(_silu(c1), wc2ab_ref[...],
                   preferred_element_type=jnp.float32)
    c2 = cji2 * scale_band
    ssb = jnp.dot(c2 * c2, g_ref[...], preferred_element_type=jnp.float32)
    robs_band = jnp.dot(robs_ref[...], rexp_ref[...],
                        preferred_element_type=jnp.float32)
    weighted = c2 * lax.rsqrt(jnp.maximum(ssb, 1e-24)) * robs_band
    lcao = lax.dot_general(weighted, t32_ref[...], (((1,), (1,)), ((), ())),
                           preferred_element_type=jnp.float32)
    ss2 = jnp.sum(lcao * lcao, axis=1, keepdims=True)
    lcao_n = lcao * lax.rsqrt(jnp.maximum(ss2, 1e-24))

    nm = jnp.dot(_silu(nm1_ref[..., :CV] + bn1_ref[...]), wn2_ref[...],
                 preferred_element_type=jnp.float32) + bn2_ref[...]
    msg = lcao_n * nm
    msg_ref[...] = jnp.concatenate(
        [msg, jnp.zeros((eb, 128 - CV), jnp.float32)], axis=1)


def _tc3(cji144, robs, agg, nm1, wc1b, wc2ab, g, rexp, t32, wt, bt, bn1,
         wn2, bn2):
    eb = EB3
    nblk = E // eb
    grid = (nblk,)
    w_spec = lambda shape: pl.BlockSpec(shape, lambda i: (0, 0))
    return pl.pallas_call(
        _tc3_body,
        grid=grid,
        in_specs=[
            pl.BlockSpec((eb, D * 16), lambda i: (i, 0)),
            pl.BlockSpec((eb, D), lambda i: (i, 0)),
            pl.BlockSpec((eb, 128), lambda i: (i, 0)),
            pl.BlockSpec((eb, 128), lambda i: (i, 0)),
            w_spec((D * 16, D * CV)), w_spec((D * CV, D * CV)),
            w_spec((D * CV, D * CV)), w_spec((D, D * CV)),
            w_spec((CV, D * CV)),
            w_spec((CV, CV)), w_spec((1, CV)), w_spec((1, CV)),
            w_spec((CV, CV)), w_spec((1, CV)),
        ],
        out_specs=pl.BlockSpec((eb, 128), lambda i: (i, 0)),
        out_shape=jax.ShapeDtypeStruct((E, 128), jnp.float32),
    )(cji144, robs, agg, nm1, wc1b, wc2ab, g, rexp, t32, wt, bt, bn1,
      wn2, bn2)


def _sc3_body(msg_hbm, ii_hbm, part_hbm,
              ii_v, rows_v, zslab_v, sp_ref):
    c = lax.axis_index("c")
    s = lax.axis_index("s")
    wid = s * NC + c

    def zrow(t, _):
        zslab_v[t, pl.ds(0, 16)] = jnp.zeros((16,), jnp.float32)
        zslab_v[t, pl.ds(16, 16)] = jnp.zeros((16,), jnp.float32)
        return 0
    lax.fori_loop(0, 125, zrow, 0)

    rows_per_tile = N // NS

    def zero_body(i, _):
        pltpu.sync_copy(zslab_v, sp_ref.at[pl.ds(s * rows_per_tile + i * 125, 125)])
        return 0
    lax.fori_loop(0, rows_per_tile // 125, zero_body, 0)
    plsc.subcore_barrier()

    n_my = 39 + jnp.where(wid < (N_ECH - 39 * NW), 1, 0)

    def scan_body(i, _):
        base = (wid + i * NW) * CH
        pltpu.sync_copy(ii_hbm.at[pl.ds(base, CH)], ii_v)
        pltpu.sync_copy(msg_hbm.at[pl.ds(base, CH), pl.ds(0, CV)], rows_v)
        pltpu.sync_copy(rows_v, sp_ref.at[ii_v], add=True)
        return 0

    lax.fori_loop(0, n_my, scan_body, 0)
    plsc.subcore_barrier()

    def wb_body(i, _):
        off = s * rows_per_tile + i * 125
        pltpu.sync_copy(sp_ref.at[pl.ds(off, 125)], rows_v.at[pl.ds(0, 125)])
        pltpu.sync_copy(rows_v.at[pl.ds(0, 125)],
                        part_hbm.at[pl.ds(c * N + off, 125), pl.ds(0, CV)])
        return 0
    lax.fori_loop(0, rows_per_tile // 125, wb_body, 0)


def _sc3(msg, idx_i):
    mesh = plsc.VectorSubcoreMesh(core_axis_name="c", subcore_axis_name="s",
                                  num_cores=NC, num_subcores=NS)
    f = pl.kernel(
        _sc3_body,
        out_type=jax.ShapeDtypeStruct((NC * N, 128), jnp.float32),
        mesh=mesh,
        compiler_params=pltpu.CompilerParams(needs_layout_passes=False,
                                             use_tc_tiling_on_sc=False),
        scratch_types=[
            pltpu.VMEM((CH,), jnp.int32),
            pltpu.VMEM((CH, CV), jnp.float32),
            pltpu.VMEM((125, CV), jnp.float32),
            pltpu.VMEM_SHARED((N, CV), jnp.float32),
        ],
    )
    return f(msg, idx_i)


def _tc4_body(x_ref, p0_ref, p1_ref, wna_ref, out_ref):
    agg = p0_ref[..., :CV] + p1_ref[..., :CV]
    out_ref[...] = x_ref[...] + jnp.dot(agg, wna_ref[...],
                                        preferred_element_type=jnp.float32)


def _tc4(x, parts, wna):
    bn = 1000
    nb = N // bn
    grid = (nb,)
    return pl.pallas_call(
        _tc4_body,
        grid=grid,
        in_specs=[
            pl.BlockSpec((bn, HID), lambda i: (i, 0)),
            pl.BlockSpec((bn, 128), lambda i: (i, 0)),
            pl.BlockSpec((bn, 128), lambda i: (i + nb, 0)),
            pl.BlockSpec((CV, HID), lambda i: (0, 0)),
        ],
        out_specs=pl.BlockSpec((bn, HID), lambda i: (i, 0)),
        out_shape=jax.ShapeDtypeStruct((N, HID), jnp.float32),
    )(x, parts, parts, wna)


def kernel(x, cji, outer_mask, robs, shbs, idx_i, idx_j, tri_idx_k,
           edge_idx_kj, edge_idx_ji, W_nb, b_nb, W_c1, W_c2, W_t, b_t,
           W_n1, b_n1, W_n2, b_n2, W_na):
    del outer_mask
    idx_i = idx_i.astype(jnp.int32)
    idx_j = idx_j.astype(jnp.int32)
    tri_idx_k = tri_idx_k.astype(jnp.int32)
    edge_idx_kj = edge_idx_kj.astype(jnp.int32)
    edge_idx_ji = edge_idx_ji.astype(jnp.int32)

    wxh, wxk = W_nb[:, :CV], W_nb[:, CV:]
    bxh, bxk = b_nb[:CV].reshape(1, CV), b_nb[CV:].reshape(1, CV)
    wc2a, wc2b = W_c2[:, :CV], W_c2[:, CV:]
    wn1a, wn1b = W_n1[:CV, :], W_n1[CV:, :]
    bt = b_t.reshape(1, CV)
    bn1 = b_n1.reshape(1, CV)
    bn2 = b_n2.reshape(1, CV)

    cji144 = cji.reshape(E, D * 16)
    ii = jnp.arange(D * CV, dtype=jnp.int32)
    dgrp = ii // CV
    eye_d = lambda a, b: (a[:, None] == b[None, :]).astype(jnp.float32)
    blkdiag16 = jnp.zeros((D * 16, D * CV), jnp.float32)
    for d in range(D):
        blkdiag16 = blkdiag16.at[d * 16:(d + 1) * 16,
                                 d * CV:(d + 1) * CV].set(W_c1)
    wc2ab = jnp.zeros((D * CV, D * CV), jnp.float32)
    wc2bb = jnp.zeros((D * CV, D * CV), jnp.float32)
    for d in range(D):
        sl = slice(d * CV, (d + 1) * CV)
        wc2ab = wc2ab.at[sl, sl].set(wc2a)
        wc2bb = wc2bb.at[sl, sl].set(wc2b)
    g = eye_d(dgrp, dgrp)
    rexp = eye_d(jnp.arange(D, dtype=jnp.int32), dgrp)
    t32 = eye_d(jnp.arange(CV, dtype=jnp.int32), ii % CV)

    sigxk, a_tab, b_tab = _tc1(x, wxh, bxh, wxk, bxk, wn1a, wn1b)
    m_table = _tc2(cji144, robs, blkdiag16, wc2bb, g, rexp)

    tbw = _sc1(m_table, shbs.reshape(T * D), sigxk, edge_idx_kj, tri_idx_k)
    nm1 = _scg(a_tab, b_tab, idx_i, idx_j)
    agg = _sc2(tbw, edge_idx_ji)

    msg = _tc3(cji144, robs, agg, nm1, blkdiag16, wc2ab, g, rexp, t32,
               W_t, bt, bn1, W_n2, bn2)
    parts = _sc3(msg, idx_i)
    return _tc4(x, parts, W_na)
